# baseline (speedup 1.0000x reference)
import functools

import jax
import jax.numpy as jnp
from jax import lax
from jax.experimental import pallas as pl
from jax.experimental.pallas import tpu as pltpu
from jax.experimental.pallas import tpu_sc as plsc

EMBED = 1024
FF = 2048
NE = 8
TOPK = 2
T = 2048
NPAIR = T * TOPK
BT = 512
NB = NPAIR // BT + NE
NPAD = NB * BT
LANES = 128


def _router_body(x_ref, rw_ref, e1_ref, e2_ref, p1_ref, p2_ref):
    logits = jnp.dot(x_ref[...], rw_ref[...], preferred_element_type=jnp.float32)
    lane = lax.broadcasted_iota(jnp.int32, (T, LANES), 1)
    neg = jnp.float32(-1e30)
    logits = jnp.where(lane < NE, logits, neg)
    m1 = jnp.max(logits, axis=1, keepdims=True)
    i1 = jnp.min(jnp.where(logits == m1, lane, LANES), axis=1, keepdims=True)
    l2 = jnp.where(lane == i1, neg, logits)
    m2 = jnp.max(l2, axis=1, keepdims=True)
    i2 = jnp.min(jnp.where(l2 == m2, lane, LANES), axis=1, keepdims=True)
    p1 = 1.0 / (1.0 + jnp.exp(m2 - m1))
    p2 = 1.0 - p1
    zero = jnp.zeros((T, LANES), jnp.int32)
    e1_ref[...] = zero + i1
    e2_ref[...] = zero + i2
    p1_ref[...] = jnp.zeros((T, LANES), jnp.float32) + p1
    p2_ref[...] = jnp.zeros((T, LANES), jnp.float32) + p2


def _router(x_flat, router_w):
    rw_pad = jnp.zeros((EMBED, LANES), jnp.float32).at[:, :NE].set(router_w)
    outs = pl.pallas_call(
        _router_body,
        out_shape=(
            jax.ShapeDtypeStruct((T, LANES), jnp.int32),
            jax.ShapeDtypeStruct((T, LANES), jnp.int32),
            jax.ShapeDtypeStruct((T, LANES), jnp.float32),
            jax.ShapeDtypeStruct((T, LANES), jnp.float32),
        ),
    )(x_flat, rw_pad)
    e1, e2, p1, p2 = (o[:, 0] for o in outs)
    return e1, e2, p1, p2


def _plan(e1, e2):
    keys = jnp.concatenate([e1, e2])
    onehot = (keys[:, None] == jnp.arange(NE, dtype=jnp.int32)[None, :]).astype(jnp.int32)
    csum = jnp.cumsum(onehot, axis=0)
    rank = jnp.sum(onehot * csum, axis=1) - 1
    counts = csum[-1]
    nblk = (counts + BT - 1) // BT
    blk_start = jnp.cumsum(nblk) - nblk
    num_used = jnp.sum(nblk)
    pos_by_pair = (jnp.sum(onehot * blk_start[None, :], axis=1) * BT + rank).astype(jnp.int32)
    blk_end = jnp.cumsum(nblk)
    bidx = jnp.arange(NB, dtype=jnp.int32)
    block_expert = jnp.clip(
        jnp.sum((blk_end[None, :] <= bidx[:, None]).astype(jnp.int32), axis=1),
        0, NE - 1).astype(jnp.int32)
    block_active = (bidx < num_used).astype(jnp.int32)
    return pos_by_pair, block_expert, block_active


NW = 32
DCH = 32
NDC = NPAIR // NW // DCH
CCH = 16
NCC = T // NW // CCH


@functools.lru_cache(maxsize=None)
def _sc_dispatch_k():
    mesh = plsc.VectorSubcoreMesh(core_axis_name="c", subcore_axis_name="s")

    @functools.partial(
        pl.kernel,
        mesh=mesh,
        out_type=jax.ShapeDtypeStruct((NPAD, EMBED), jnp.float32),
        scratch_types=[
            pltpu.VMEM((2, DCH), jnp.int32),
            pltpu.VMEM((2, DCH), jnp.int32),
            pltpu.VMEM((2, DCH, EMBED), jnp.float32),
            pltpu.SemaphoreType.DMA,
            pltpu.SemaphoreType.DMA,
            pltpu.SemaphoreType.DMA,
            pltpu.SemaphoreType.DMA,
        ],
    )
    def k(x_hbm, tok_hbm, pos_hbm, xs_hbm, tok_v, pos_v, rows_v,
          g0, g1, s0, s1):
        wid = lax.axis_index("s") * 2 + lax.axis_index("c")
        per_w = NPAIR // NW
        gsem = (g0, g1)
        ssem = (s0, s1)

        def start_gather(c):
            s = c % 2
            base = wid * per_w + c * DCH
            pltpu.sync_copy(tok_hbm.at[pl.ds(base, DCH)], tok_v.at[s])
            pltpu.sync_copy(pos_hbm.at[pl.ds(base, DCH)], pos_v.at[s])
            return pltpu.async_copy(x_hbm.at[tok_v.at[s]], rows_v.at[s], gsem[s])

        gh = [start_gather(0), start_gather(1)]
        sh = [None, None]
        for c in range(NDC):
            s = c % 2
            gh[s].wait()
            sh[s] = pltpu.async_copy(rows_v.at[s], xs_hbm.at[pos_v.at[s]], ssem[s])
            if c + 2 < NDC:
                sh[s].wait()
                gh[s] = start_gather(c + 2)
        for c in range(max(0, NDC - 2), NDC):
            sh[c % 2].wait()

    return k


@functools.lru_cache(maxsize=None)
def _sc_combine_k():
    mesh = plsc.VectorSubcoreMesh(core_axis_name="c", subcore_axis_name="s")

    @functools.partial(
        pl.kernel,
        mesh=mesh,
        out_type=(
            jax.ShapeDtypeStruct((T, EMBED), jnp.float32),
            jax.ShapeDtypeStruct((T, EMBED), jnp.float32),
        ),
        scratch_types=[
            pltpu.VMEM((2, CCH), jnp.int32),
            pltpu.VMEM((2, CCH), jnp.int32),
            pltpu.VMEM((2, CCH, EMBED), jnp.float32),
            pltpu.VMEM((2, CCH, EMBED), jnp.float32),
            pltpu.SemaphoreType.DMA,
            pltpu.SemaphoreType.DMA,
            pltpu.SemaphoreType.DMA,
            pltpu.SemaphoreType.DMA,
            pltpu.SemaphoreType.DMA,
            pltpu.SemaphoreType.DMA,
            pltpu.SemaphoreType.DMA,
            pltpu.SemaphoreType.DMA,
        ],
    )
    def k(y_hbm, i0_hbm, i1_hbm, o0_hbm, o1_hbm, i0_v, i1_v, a_v, b_v,
          ga0, ga1, gb0, gb1, wa0, wa1, wb0, wb1):
        wid = lax.axis_index("s") * 2 + lax.axis_index("c")
        per_w = T // NW
        gasem = (ga0, ga1)
        gbsem = (gb0, gb1)
        wasem = (wa0, wa1)
        wbsem = (wb0, wb1)

        def start_gathers(c):
            s = c % 2
            base = wid * per_w + c * CCH
            pltpu.sync_copy(i0_hbm.at[pl.ds(base, CCH)], i0_v.at[s])
            pltpu.sync_copy(i1_hbm.at[pl.ds(base, CCH)], i1_v.at[s])
            return (pltpu.async_copy(y_hbm.at[i0_v.at[s]], a_v.at[s], gasem[s]),
                    pltpu.async_copy(y_hbm.at[i1_v.at[s]], b_v.at[s], gbsem[s]))

        gh = [start_gathers(0), start_gathers(1)]
        wh = [None, None]
        for c in range(NCC):
            s = c % 2
            base = wid * per_w + c * CCH
            gh[s][0].wait()
            gh[s][1].wait()
            wh[s] = (pltpu.async_copy(a_v.at[s], o0_hbm.at[pl.ds(base, CCH)], wasem[s]),
                     pltpu.async_copy(b_v.at[s], o1_hbm.at[pl.ds(base, CCH)], wbsem[s]))
            if c + 2 < NCC:
                wh[s][0].wait()
                wh[s][1].wait()
                gh[s] = start_gathers(c + 2)
        for c in range(max(0, NCC - 2), NCC):
            wh[c % 2][0].wait()
            wh[c % 2][1].wait()

    return k


def _sc_gather(x_flat, tok_pair, pos_by_pair):
    return _sc_dispatch_k()(x_flat, tok_pair, pos_by_pair)


def _sc_combine(y, i0, i1):
    return _sc_combine_k()(y, i0, i1)


def _gemm_body(be_ref, ba_ref, x_ref, gw_ref, uw_ref, dw_ref, y_ref):
    b = pl.program_id(0)
    f = pl.program_id(1)

    @pl.when(ba_ref[b] == 1)
    def _():
        x = x_ref[...]
        g = jnp.dot(x, gw_ref[0], preferred_element_type=jnp.float32)
        u = jnp.dot(x, uw_ref[0], preferred_element_type=jnp.float32)
        h = (g * lax.logistic(g)) * u
        part = jnp.dot(h, dw_ref[0], preferred_element_type=jnp.float32)

        @pl.when(f == 0)
        def _():
            y_ref[...] = part

        @pl.when(f != 0)
        def _():
            y_ref[...] += part


NF = 2


def _grouped_gemm(x_s, gate_w, up_w, down_w, block_expert, block_active):
    def _rowmap(b, f, be, ba):
        return (jnp.where(ba[b] == 1, b, NB - 1), 0)

    grid_spec = pltpu.PrefetchScalarGridSpec(
        num_scalar_prefetch=2,
        grid=(NB, NF),
        in_specs=[
            pl.BlockSpec((BT, EMBED), _rowmap),
            pl.BlockSpec((1, EMBED, FF // NF), lambda b, f, be, ba: (be[b], 0, f)),
            pl.BlockSpec((1, EMBED, FF // NF), lambda b, f, be, ba: (be[b], 0, f)),
            pl.BlockSpec((1, FF // NF, EMBED), lambda b, f, be, ba: (be[b], f, 0)),
        ],
        out_specs=pl.BlockSpec((BT, EMBED), _rowmap),
    )
    return pl.pallas_call(
        _gemm_body,
        grid_spec=grid_spec,
        out_shape=jax.ShapeDtypeStruct((NPAD, EMBED), jnp.float32),
        compiler_params=pltpu.CompilerParams(
            dimension_semantics=("arbitrary", "arbitrary"),
        ),
    )(block_expert, block_active, x_s, gate_w, up_w, down_w)


def kernel(x, router_w, gate_w, up_w, down_w):
    B, S, D = x.shape
    x_flat = x.reshape(T, D)
    e1, e2, p1, p2 = _router(x_flat, router_w)
    pos_by_pair, block_expert, block_active = _plan(e1, e2)
    tok_pair = jnp.tile(jnp.arange(T, dtype=jnp.int32), 2)
    x_s = _sc_gather(x_flat, tok_pair, pos_by_pair)
    y = _grouped_gemm(x_s, gate_w, up_w, down_w, block_expert, block_active)
    g0, g1 = _sc_combine(y, pos_by_pair[:T], pos_by_pair[T:])
    out = p1[:, None] * g0 + p2[:, None] * g1
    return out.reshape(B, S, D)


# speedup vs baseline: 1.1379x; 1.1379x over previous
import functools

import jax
import jax.numpy as jnp
from jax import lax
from jax.experimental import pallas as pl
from jax.experimental.pallas import tpu as pltpu
from jax.experimental.pallas import tpu_sc as plsc

EMBED = 1024
FF = 2048
NE = 8
TOPK = 2
T = 2048
NPAIR = T * TOPK
BT = 256
NB = NPAIR // BT + NE
NPAD = NB * BT
LANES = 128


_NCHUNK = T // LANES


def _router_body(x_ref, rw_ref, pos1_ref, pos2_ref, p1_ref, p2_ref,
                 beba_ref, cs1_ref, cs2_ref):
    logits = jnp.dot(x_ref[...], rw_ref[...], preferred_element_type=jnp.float32)
    lane = lax.broadcasted_iota(jnp.int32, (T, LANES), 1)
    neg = jnp.float32(-1e30)
    logits = jnp.where(lane < NE, logits, neg)
    m1 = jnp.max(logits, axis=1, keepdims=True)
    i1 = jnp.min(jnp.where(logits == m1, lane, LANES), axis=1, keepdims=True)
    l2 = jnp.where(lane == i1, neg, logits)
    m2 = jnp.max(l2, axis=1, keepdims=True)
    i2 = jnp.min(jnp.where(l2 == m2, lane, LANES), axis=1, keepdims=True)
    p1 = 1.0 / (1.0 + jnp.exp(m2 - m1))
    p1_ref[...] = jnp.zeros((T, LANES), jnp.float32) + p1
    p2_ref[...] = jnp.zeros((T, LANES), jnp.float32) + (1.0 - p1)

    oh1 = (lane == i1).astype(jnp.float32)
    oh2 = (lane == i2).astype(jnp.float32)
    ri = lax.broadcasted_iota(jnp.int32, (LANES, LANES), 0)
    ci = lax.broadcasted_iota(jnp.int32, (LANES, LANES), 1)
    ltri = (ci <= ri).astype(jnp.float32)
    run = jnp.zeros((1, LANES), jnp.float32)
    for oh, cs_ref in ((oh1, cs1_ref), (oh2, cs2_ref)):
        for c in range(_NCHUNK):
            blk = lax.slice(oh, (c * LANES, 0), ((c + 1) * LANES, LANES))
            cs = jnp.dot(ltri, blk, preferred_element_type=jnp.float32) + run
            cs_ref[pl.ds(c * LANES, LANES), :] = cs
            run = lax.slice(cs, (LANES - 1, 0), (LANES, LANES))
    counts = run
    nblk = (counts.astype(jnp.int32) + BT - 1) >> 8
    nblk_f = nblk.astype(jnp.float32)
    ustri = (ri < ci).astype(jnp.float32)
    bs = jnp.dot(nblk_f, ustri, preferred_element_type=jnp.float32)
    num_used = jnp.sum(nblk_f)
    base1 = bs * jnp.float32(BT) - 1.0
    pos1 = jnp.sum(oh1 * (cs1_ref[...] + base1), axis=1, keepdims=True)
    pos2 = jnp.sum(oh2 * (cs2_ref[...] + base1), axis=1, keepdims=True)
    pos1_ref[...] = jnp.zeros((T, LANES), jnp.int32) + pos1.astype(jnp.int32)
    pos2_ref[...] = jnp.zeros((T, LANES), jnp.int32) + pos2.astype(jnp.int32)
    bidx = lax.broadcasted_iota(jnp.int32, (32, LANES), 0).astype(jnp.float32)
    bend = jnp.dot(nblk_f, ustri + (ri == ci).astype(jnp.float32),
                   preferred_element_type=jnp.float32)
    laneok = (lax.broadcasted_iota(jnp.int32, (32, LANES), 1) < NE)
    be = jnp.sum(jnp.where(laneok & (bend <= bidx), 1.0, 0.0), axis=1, keepdims=True)
    be = jnp.minimum(be, jnp.float32(NE - 1))
    ba = (bidx[:, 0:1] < num_used).astype(jnp.float32)
    beba_ref[...] = (jnp.zeros((32, LANES), jnp.float32)
                     + be + 16.0 * ba).astype(jnp.int32)


def _router_plan(x_flat, router_w):
    rw_pad = jnp.zeros((EMBED, LANES), jnp.float32).at[:, :NE].set(router_w)
    pos1o, pos2o, p1o, p2o, bebao = pl.pallas_call(
        _router_body,
        out_shape=(
            jax.ShapeDtypeStruct((T, LANES), jnp.int32),
            jax.ShapeDtypeStruct((T, LANES), jnp.int32),
            jax.ShapeDtypeStruct((T, LANES), jnp.float32),
            jax.ShapeDtypeStruct((T, LANES), jnp.float32),
            jax.ShapeDtypeStruct((32, LANES), jnp.int32),
        ),
        scratch_shapes=[
            pltpu.VMEM((T, LANES), jnp.float32),
            pltpu.VMEM((T, LANES), jnp.float32),
        ],
    )(x_flat, rw_pad)
    i0 = pos1o[:, 0]
    i1 = pos2o[:, 0]
    beba = bebao[:NB, 0]
    block_expert = beba & 15
    block_active = beba >> 4
    return i0, i1, p1o[:, 0], p2o[:, 0], block_expert, block_active


NW = 32
DCH = 32
NDC = NPAIR // NW // DCH
CCH = 16
NCC = T // NW // CCH


@functools.lru_cache(maxsize=None)
def _sc_dispatch_k():
    mesh = plsc.VectorSubcoreMesh(core_axis_name="c", subcore_axis_name="s")

    @functools.partial(
        pl.kernel,
        mesh=mesh,
        out_type=jax.ShapeDtypeStruct((NPAD, EMBED), jnp.float32),
        scratch_types=[
            pltpu.VMEM((2, DCH), jnp.int32),
            pltpu.VMEM((2, DCH), jnp.int32),
            pltpu.VMEM((2, DCH, EMBED), jnp.float32),
            pltpu.SemaphoreType.DMA,
            pltpu.SemaphoreType.DMA,
            pltpu.SemaphoreType.DMA,
            pltpu.SemaphoreType.DMA,
        ],
    )
    def k(x_hbm, tok_hbm, pos_hbm, xs_hbm, tok_v, pos_v, rows_v,
          g0, g1, s0, s1):
        wid = lax.axis_index("s") * 2 + lax.axis_index("c")
        per_w = NPAIR // NW
        gsem = (g0, g1)
        ssem = (s0, s1)

        def start_gather(c):
            s = c % 2
            base = wid * per_w + c * DCH
            pltpu.sync_copy(tok_hbm.at[pl.ds(base, DCH)], tok_v.at[s])
            pltpu.sync_copy(pos_hbm.at[pl.ds(base, DCH)], pos_v.at[s])
            return pltpu.async_copy(x_hbm.at[tok_v.at[s]], rows_v.at[s], gsem[s])

        gh = [start_gather(0), start_gather(1)]
        sh = [None, None]
        for c in range(NDC):
            s = c % 2
            gh[s].wait()
            sh[s] = pltpu.async_copy(rows_v.at[s], xs_hbm.at[pos_v.at[s]], ssem[s])
            if c + 2 < NDC:
                sh[s].wait()
                gh[s] = start_gather(c + 2)
        for c in range(max(0, NDC - 2), NDC):
            sh[c % 2].wait()

    return k


@functools.lru_cache(maxsize=None)
def _sc_combine_k():
    mesh = plsc.VectorSubcoreMesh(core_axis_name="c", subcore_axis_name="s")

    @functools.partial(
        pl.kernel,
        mesh=mesh,
        out_type=(
            jax.ShapeDtypeStruct((T, EMBED), jnp.float32),
            jax.ShapeDtypeStruct((T, EMBED), jnp.float32),
        ),
        scratch_types=[
            pltpu.VMEM((2, CCH), jnp.int32),
            pltpu.VMEM((2, CCH), jnp.int32),
            pltpu.VMEM((2, CCH, EMBED), jnp.float32),
            pltpu.VMEM((2, CCH, EMBED), jnp.float32),
            pltpu.SemaphoreType.DMA,
            pltpu.SemaphoreType.DMA,
            pltpu.SemaphoreType.DMA,
            pltpu.SemaphoreType.DMA,
            pltpu.SemaphoreType.DMA,
            pltpu.SemaphoreType.DMA,
            pltpu.SemaphoreType.DMA,
            pltpu.SemaphoreType.DMA,
        ],
    )
    def k(y_hbm, i0_hbm, i1_hbm, o0_hbm, o1_hbm, i0_v, i1_v, a_v, b_v,
          ga0, ga1, gb0, gb1, wa0, wa1, wb0, wb1):
        wid = lax.axis_index("s") * 2 + lax.axis_index("c")
        per_w = T // NW
        gasem = (ga0, ga1)
        gbsem = (gb0, gb1)
        wasem = (wa0, wa1)
        wbsem = (wb0, wb1)

        def start_gathers(c):
            s = c % 2
            base = wid * per_w + c * CCH
            pltpu.sync_copy(i0_hbm.at[pl.ds(base, CCH)], i0_v.at[s])
            pltpu.sync_copy(i1_hbm.at[pl.ds(base, CCH)], i1_v.at[s])
            return (pltpu.async_copy(y_hbm.at[i0_v.at[s]], a_v.at[s], gasem[s]),
                    pltpu.async_copy(y_hbm.at[i1_v.at[s]], b_v.at[s], gbsem[s]))

        gh = [start_gathers(0), start_gathers(1)]
        wh = [None, None]
        for c in range(NCC):
            s = c % 2
            base = wid * per_w + c * CCH
            gh[s][0].wait()
            gh[s][1].wait()
            wh[s] = (pltpu.async_copy(a_v.at[s], o0_hbm.at[pl.ds(base, CCH)], wasem[s]),
                     pltpu.async_copy(b_v.at[s], o1_hbm.at[pl.ds(base, CCH)], wbsem[s]))
            if c + 2 < NCC:
                wh[s][0].wait()
                wh[s][1].wait()
                gh[s] = start_gathers(c + 2)
        for c in range(max(0, NCC - 2), NCC):
            wh[c % 2][0].wait()
            wh[c % 2][1].wait()

    return k


def _sc_gather(x_flat, tok_pair, pos_by_pair):
    return _sc_dispatch_k()(x_flat, tok_pair, pos_by_pair)


def _sc_combine(y, i0, i1):
    return _sc_combine_k()(y, i0, i1)


def _gemm_body(be_ref, ba_ref, x_ref, gw_ref, uw_ref, dw_ref, y_ref):
    b = pl.program_id(0)

    @pl.when(ba_ref[b] == 1)
    def _():
        x = x_ref[...]
        g = jnp.dot(x, gw_ref[0], preferred_element_type=jnp.float32)
        u = jnp.dot(x, uw_ref[0], preferred_element_type=jnp.float32)
        h = (g * lax.logistic(g)) * u
        y_ref[...] = jnp.dot(h, dw_ref[0], preferred_element_type=jnp.float32)


def _grouped_gemm(x_s, gate_w, up_w, down_w, block_expert, block_active):
    def _rowmap(b, be, ba):
        return (jnp.where(ba[b] == 1, b, NB - 1), 0)

    grid_spec = pltpu.PrefetchScalarGridSpec(
        num_scalar_prefetch=2,
        grid=(NB,),
        in_specs=[
            pl.BlockSpec((BT, EMBED), _rowmap),
            pl.BlockSpec((1, EMBED, FF), lambda b, be, ba: (be[b], 0, 0)),
            pl.BlockSpec((1, EMBED, FF), lambda b, be, ba: (be[b], 0, 0)),
            pl.BlockSpec((1, FF, EMBED), lambda b, be, ba: (be[b], 0, 0)),
        ],
        out_specs=pl.BlockSpec((BT, EMBED), _rowmap),
    )
    return pl.pallas_call(
        _gemm_body,
        grid_spec=grid_spec,
        out_shape=jax.ShapeDtypeStruct((NPAD, EMBED), jnp.float32),
        compiler_params=pltpu.CompilerParams(
            dimension_semantics=("arbitrary",),
        ),
    )(block_expert, block_active, x_s, gate_w, up_w, down_w)


def kernel(x, router_w, gate_w, up_w, down_w):
    B, S, D = x.shape
    x_flat = x.reshape(T, D)
    i0, i1, p1, p2, block_expert, block_active = _router_plan(x_flat, router_w)
    pos_by_pair = jnp.concatenate([i0, i1])
    tok_pair = jnp.tile(jnp.arange(T, dtype=jnp.int32), 2)
    x_s = _sc_gather(x_flat, tok_pair, pos_by_pair)
    y = _grouped_gemm(x_s, gate_w, up_w, down_w, block_expert, block_active)
    g0, g1 = _sc_combine(y, i0, i1)
    out = p1[:, None] * g0 + p2[:, None] * g1
    return out.reshape(B, S, D)
